# baseline (speedup 1.0000x reference)
import functools

import numpy as np

import jax
import jax.numpy as jnp
from jax import lax
from jax.experimental import pallas as pl
from jax.experimental.pallas import tpu as pltpu

_D = 64
_S = 10
_TILE = 64

_KS0 = np.uint32(0)
_KS1 = np.uint32(42)
_KS2 = np.uint32(_KS0 ^ _KS1 ^ np.uint32(0x1BD11BDA))
_ROT_A = (13, 15, 26, 6)
_ROT_B = (17, 29, 16, 24)
_INJ = (
    (_KS1, np.uint32(_KS2 + np.uint32(1))),
    (_KS2, np.uint32(_KS0 + np.uint32(2))),
    (_KS0, np.uint32(_KS1 + np.uint32(3))),
    (_KS1, np.uint32(_KS2 + np.uint32(4))),
    (_KS2, np.uint32(_KS0 + np.uint32(5))),
)
_TINY = np.float32(np.finfo(np.float32).tiny)


def _rotl(v, r):
    return lax.shift_left(v, np.uint32(r)) | lax.shift_right_logical(
        v, np.uint32(32 - r))


def _threefry_bits(cnt):
    x0 = jnp.full_like(cnt, _KS0)
    x1 = cnt + _KS1
    for g in range(5):
        rots = _ROT_A if g % 2 == 0 else _ROT_B
        for r in rots:
            x0 = x0 + x1
            x1 = _rotl(x1, r)
            x1 = x1 ^ x0
        a, b = _INJ[g]
        x0 = x0 + a
        x1 = x1 + b
    return x0 ^ x1


def _vq_body(n, K, x_ref, e_ref, q_ref, s_ref):
    i = pl.program_id(0)
    x = x_ref[...]
    e = e_ref[...]
    prod = lax.dot_general(x, e, (((1,), (1,)), ((), ())),
                           preferred_element_type=jnp.float32)
    xn = jnp.sum(x * x, axis=1, keepdims=True)
    ones = jnp.ones((1, _D), jnp.float32)
    en = lax.dot_general(ones, e * e, (((1,), (1,)), ((), ())),
                         preferred_element_type=jnp.float32)
    dists = (xn + en - (prod + prod)) * np.float32(1.0 / K)
    c = jnp.exp(dists)

    col = lax.broadcasted_iota(jnp.int32, (_TILE, K), 1)
    rowu = lax.broadcasted_iota(jnp.uint32, (_TILE, K), 0)
    iu = lax.convert_element_type(i, jnp.uint32)
    base = (iu * np.uint32(_TILE * K)
            + rowu * np.uint32(K)
            + col.astype(jnp.uint32))

    idxs = []
    for s in range(_S):
        cnt = base + np.uint32(s * n * K)
        bits = _threefry_bits(cnt)
        fb = lax.shift_right_logical(bits, np.uint32(9)) | np.uint32(0x3F800000)
        f = lax.bitcast_convert_type(fb, jnp.float32) - np.float32(1.0)
        u = jnp.maximum(f + _TINY, _TINY)
        t = -jnp.log(u) * c
        idx = jnp.argmin(t, axis=1).astype(jnp.int32)
        idxs.append(idx[:, None])
    samples = jnp.concatenate(idxs, axis=1)
    s_ref[...] = samples

    counts = functools.reduce(
        lambda acc, id_: acc + (id_ == col).astype(jnp.float32),
        idxs, jnp.zeros((_TILE, K), jnp.float32))
    q = lax.dot_general(counts, e, (((1,), (0,)), ((), ())),
                        preferred_element_type=jnp.float32) / np.float32(_S)
    q_ref[...] = x + (q - x)


def kernel(inputs, embeds):
    size = inputs.shape
    x = inputs.reshape(-1, _D)
    n = x.shape[0]
    K = embeds.shape[0]
    grid = (n // _TILE,)
    body = functools.partial(_vq_body, n, K)
    q, s = pl.pallas_call(
        body,
        grid=grid,
        in_specs=[
            pl.BlockSpec((_TILE, _D), lambda i: (i, 0)),
            pl.BlockSpec((K, _D), lambda i: (0, 0)),
        ],
        out_specs=[
            pl.BlockSpec((_TILE, _D), lambda i: (i, 0)),
            pl.BlockSpec((_TILE, _S), lambda i: (i, 0)),
        ],
        out_shape=[
            jax.ShapeDtypeStruct((n, _D), jnp.float32),
            jax.ShapeDtypeStruct((n, _S), jnp.int32),
        ],
        compiler_params=pltpu.CompilerParams(
            dimension_semantics=("parallel",)),
    )(x, embeds)
    return (q.reshape(size), s.reshape(size[:-1] + (_S,)))
